# baseline (speedup 1.0000x reference)
import math

import jax
import jax.numpy as jnp
from jax.experimental import pallas as pl
from jax.experimental.pallas import tpu as pltpu

BATCH = 16
NUM_Q_HEADS = 32
NUM_KV_HEADS = 8
HEAD_DIM = 128
PAGE_SIZE = 16
ALL_NUM_PAGES = 2048
GROUPS = NUM_Q_HEADS // NUM_KV_HEADS

PAGES_PER_CHUNK = 32
CHUNK_TOKENS = PAGES_PER_CHUNK * PAGE_SIZE
NSLOTS = 6
MAX_CHUNKS = ALL_NUM_PAGES // PAGES_PER_CHUNK + BATCH

NEG_INF = -1e30


def _attn_kernel(
    indptr_ref,
    indices_ref,
    lastlen_ref,
    q_ref,
    kv_hbm_ref,
    out_ref,
    wb_ref,
    wc_ref,
    kv_buf,
    s_ref,
    pv_ref,
    m_ref,
    l_ref,
    acc_ref,
    sems,
):
    kv_buf[...] = jnp.zeros_like(kv_buf)
    out_ref[...] = jnp.zeros_like(out_ref)

    def per_batch(b, total):
        n_pages = indptr_ref[b + 1] - indptr_ref[b]
        num_chunks = (n_pages + PAGES_PER_CHUNK - 1) // PAGES_PER_CHUNK

        def per_chunk(c, tot):
            wb_ref[tot] = b
            wc_ref[tot] = c
            return tot + 1

        return jax.lax.fori_loop(0, num_chunks, per_chunk, total)

    total_chunks = jax.lax.fori_loop(0, BATCH, per_batch, 0)

    def n_valid(b, c):
        n_pages = indptr_ref[b + 1] - indptr_ref[b]
        return jnp.minimum(n_pages - c * PAGES_PER_CHUNK, PAGES_PER_CHUNK)

    def one_copy(b, c, slot, j):
        idx = indices_ref[indptr_ref[b] + c * PAGES_PER_CHUNK + j]
        return pltpu.make_async_copy(
            kv_hbm_ref.at[idx], kv_buf.at[slot, j], sems.at[slot, j])

    def issue(g):
        b = wb_ref[g]
        c = wc_ref[g]
        slot = jax.lax.rem(g, NSLOTS)
        jax.lax.fori_loop(
            0, n_valid(b, c),
            lambda j, car: (one_copy(b, c, slot, j).start(), car)[1], 0)

    def wait(g):
        b = wb_ref[g]
        c = wc_ref[g]
        slot = jax.lax.rem(g, NSLOTS)
        jax.lax.fori_loop(
            0, n_valid(b, c),
            lambda j, car: (one_copy(b, c, slot, j).wait(), car)[1], 0)

    @pl.when(total_chunks > 0)
    def _():
        for la in range(NSLOTS - 1):
            @pl.when(la < total_chunks)
            def _():
                issue(la)

        def body(g, carry):
            b = wb_ref[g]
            c = wc_ref[g]
            slot = jax.lax.rem(g, NSLOTS)

            @pl.when(g + NSLOTS - 1 < total_chunks)
            def _():
                issue(g + NSLOTS - 1)

            @pl.when(c == 0)
            def _():
                m_ref[...] = jnp.full_like(m_ref, NEG_INF)
                l_ref[...] = jnp.zeros_like(l_ref)
                acc_ref[...] = jnp.zeros_like(acc_ref)

            wait(g)

            n_pages = indptr_ref[b + 1] - indptr_ref[b]
            seq_len = (n_pages - 1) * PAGE_SIZE + lastlen_ref[b]
            pos = c * CHUNK_TOKENS + jax.lax.broadcasted_iota(
                jnp.int32, (1, CHUNK_TOKENS), 1)
            tok_valid = pos < seq_len

            qb = q_ref[b]
            for h in range(NUM_KV_HEADS):
                kh = kv_buf[slot, :, 0, h].reshape(CHUNK_TOKENS, HEAD_DIM)
                rows = slice(h * GROUPS, (h + 1) * GROUPS)
                s_ref[rows, :] = jax.lax.dot_general(
                    qb[rows, :], kh, (((1,), (1,)), ((), ())),
                    preferred_element_type=jnp.float32)

            s = jnp.where(tok_valid, s_ref[...], NEG_INF)
            m_old = m_ref[...]
            m_cur = jnp.max(s, axis=1, keepdims=True)
            m_new = jnp.maximum(m_old, m_cur)
            s_ref[...] = jnp.exp(s - m_new[:, 0:1])
            alpha = jnp.exp(m_old - m_new)
            l_ref[...] = l_ref[...] * alpha + \
                jnp.sum(s_ref[...], axis=1, keepdims=True)
            m_ref[...] = m_new

            for h in range(NUM_KV_HEADS):
                vh = kv_buf[slot, :, 1, h].reshape(CHUNK_TOKENS, HEAD_DIM)
                rows = slice(h * GROUPS, (h + 1) * GROUPS)
                pv_ref[rows, :] = jax.lax.dot_general(
                    s_ref[rows, :], vh, (((1,), (0,)), ((), ())),
                    preferred_element_type=jnp.float32)

            acc_ref[...] = acc_ref[...] * alpha + pv_ref[...]

            num_chunks_b = (n_pages + PAGES_PER_CHUNK - 1) // PAGES_PER_CHUNK

            @pl.when(c + 1 == num_chunks_b)
            def _():
                l = l_ref[...]
                out_ref[b] = jnp.where(l > 0, acc_ref[...] / l, 0.0)

            return carry

        jax.lax.fori_loop(0, total_chunks, body, 0)


@jax.jit
def kernel(q, paged_kv_cache, kv_page_indptr, kv_page_indices,
           kv_last_page_len):
    batch, num_q_heads, _, head_dim = q.shape
    q2 = q.reshape(batch, num_q_heads, head_dim) * (1.0 / math.sqrt(head_dim))

    grid_spec = pltpu.PrefetchScalarGridSpec(
        num_scalar_prefetch=3,
        grid=(1,),
        in_specs=[
            pl.BlockSpec(memory_space=pltpu.MemorySpace.VMEM),
            pl.BlockSpec(memory_space=pltpu.MemorySpace.HBM),
        ],
        out_specs=pl.BlockSpec(memory_space=pltpu.MemorySpace.VMEM),
        scratch_shapes=[
            pltpu.MemorySpace.SMEM((MAX_CHUNKS,), jnp.int32),
            pltpu.MemorySpace.SMEM((MAX_CHUNKS,), jnp.int32),
            pltpu.MemorySpace.VMEM(
                (NSLOTS, PAGES_PER_CHUNK, 2, NUM_KV_HEADS, PAGE_SIZE, HEAD_DIM),
                jnp.float32),
            pltpu.MemorySpace.VMEM((NUM_Q_HEADS, CHUNK_TOKENS), jnp.float32),
            pltpu.MemorySpace.VMEM((NUM_Q_HEADS, HEAD_DIM), jnp.float32),
            pltpu.MemorySpace.VMEM((NUM_Q_HEADS, 128), jnp.float32),
            pltpu.MemorySpace.VMEM((NUM_Q_HEADS, 128), jnp.float32),
            pltpu.MemorySpace.VMEM((NUM_Q_HEADS, HEAD_DIM), jnp.float32),
            pltpu.SemaphoreType.DMA((NSLOTS, PAGES_PER_CHUNK)),
        ],
    )
    out = pl.pallas_call(
        _attn_kernel,
        grid_spec=grid_spec,
        out_shape=jax.ShapeDtypeStruct((batch, num_q_heads, head_dim),
                                       jnp.float32),
    )(kv_page_indptr, kv_page_indices, kv_last_page_len,
      q2, paged_kv_cache)
    return out.reshape(batch, num_q_heads, 1, head_dim)


# speedup vs baseline: 1.0381x; 1.0381x over previous
import math

import jax
import jax.numpy as jnp
from jax.experimental import pallas as pl
from jax.experimental.pallas import tpu as pltpu

BATCH = 16
NUM_Q_HEADS = 32
NUM_KV_HEADS = 8
HEAD_DIM = 128
PAGE_SIZE = 16
ALL_NUM_PAGES = 2048
GROUPS = NUM_Q_HEADS // NUM_KV_HEADS

PAGES_PER_CHUNK = 32
CHUNK_TOKENS = PAGES_PER_CHUNK * PAGE_SIZE
NSLOTS = 4
MAX_CHUNKS = ALL_NUM_PAGES // PAGES_PER_CHUNK + BATCH

NEG_INF = -1e30


def _attn_kernel(
    indptr_ref,
    indices_ref,
    lastlen_ref,
    q_ref,
    kv_hbm_ref,
    out_ref,
    wb_ref,
    wc_ref,
    kv_buf,
    s_ref,
    pv_ref,
    m_ref,
    l_ref,
    acc_ref,
    sems,
):
    kv_buf[...] = jnp.zeros_like(kv_buf)
    out_ref[...] = jnp.zeros_like(out_ref)

    def per_batch(b, total):
        n_pages = indptr_ref[b + 1] - indptr_ref[b]
        num_chunks = (n_pages + PAGES_PER_CHUNK - 1) // PAGES_PER_CHUNK

        def per_chunk(c, tot):
            wb_ref[tot] = b
            wc_ref[tot] = c
            return tot + 1

        return jax.lax.fori_loop(0, num_chunks, per_chunk, total)

    total_chunks = jax.lax.fori_loop(0, BATCH, per_batch, 0)

    def n_valid(b, c):
        n_pages = indptr_ref[b + 1] - indptr_ref[b]
        return jnp.minimum(n_pages - c * PAGES_PER_CHUNK, PAGES_PER_CHUNK)

    def one_copy(base, slot, j):
        idx = indices_ref[base + j]
        return pltpu.make_async_copy(
            kv_hbm_ref.at[idx], kv_buf.at[slot, j], sems.at[slot, j])

    def issue(g):
        b = wb_ref[g]
        c = wc_ref[g]
        slot = jax.lax.rem(g, NSLOTS)
        base = indptr_ref[b] + c * PAGES_PER_CHUNK
        jax.lax.fori_loop(
            0, n_valid(b, c),
            lambda j, car: (one_copy(base, slot, j).start(), car)[1], 0)

    def wait(g):
        b = wb_ref[g]
        c = wc_ref[g]
        slot = jax.lax.rem(g, NSLOTS)
        base = indptr_ref[b] + c * PAGES_PER_CHUNK
        jax.lax.fori_loop(
            0, n_valid(b, c),
            lambda j, car: (one_copy(base, slot, j).wait(), car)[1], 0)

    @pl.when(total_chunks > 0)
    def _():
        for la in range(NSLOTS - 1):
            @pl.when(la < total_chunks)
            def _():
                issue(la)

        def body(g, carry):
            b = wb_ref[g]
            c = wc_ref[g]
            slot = jax.lax.rem(g, NSLOTS)

            @pl.when(g + NSLOTS - 1 < total_chunks)
            def _():
                issue(g + NSLOTS - 1)

            @pl.when(c == 0)
            def _():
                m_ref[...] = jnp.full_like(m_ref, NEG_INF)
                l_ref[...] = jnp.zeros_like(l_ref)
                acc_ref[...] = jnp.zeros_like(acc_ref)

            wait(g)

            n_pages = indptr_ref[b + 1] - indptr_ref[b]
            seq_len = (n_pages - 1) * PAGE_SIZE + lastlen_ref[b]
            pos = c * CHUNK_TOKENS + jax.lax.broadcasted_iota(
                jnp.int32, (1, CHUNK_TOKENS), 1)
            tok_valid = pos < seq_len

            qb = q_ref[b]
            for h in range(NUM_KV_HEADS):
                kh = kv_buf[slot, :, 0, h].reshape(CHUNK_TOKENS, HEAD_DIM)
                rows = slice(h * GROUPS, (h + 1) * GROUPS)
                s_ref[rows, :] = jax.lax.dot_general(
                    qb[rows, :], kh, (((1,), (1,)), ((), ())),
                    preferred_element_type=jnp.float32)

            s = jnp.where(tok_valid, s_ref[...], NEG_INF)
            m_old = m_ref[...]
            m_cur = jnp.max(s, axis=1, keepdims=True)
            m_new = jnp.maximum(m_old, m_cur)
            s_ref[...] = jnp.exp(s - m_new[:, 0:1])
            alpha = jnp.exp(m_old - m_new)
            l_ref[...] = l_ref[...] * alpha + \
                jnp.sum(s_ref[...], axis=1, keepdims=True)
            m_ref[...] = m_new

            for h in range(NUM_KV_HEADS):
                vh = kv_buf[slot, :, 1, h].reshape(CHUNK_TOKENS, HEAD_DIM)
                rows = slice(h * GROUPS, (h + 1) * GROUPS)
                pv_ref[rows, :] = jax.lax.dot_general(
                    s_ref[rows, :], vh, (((1,), (0,)), ((), ())),
                    preferred_element_type=jnp.float32)

            acc_ref[...] = acc_ref[...] * alpha + pv_ref[...]

            num_chunks_b = (n_pages + PAGES_PER_CHUNK - 1) // PAGES_PER_CHUNK

            @pl.when(c + 1 == num_chunks_b)
            def _():
                l = l_ref[...]
                out_ref[b] = jnp.where(l > 0, acc_ref[...] / l, 0.0)

            return carry

        jax.lax.fori_loop(0, total_chunks, body, 0)


@jax.jit
def kernel(q, paged_kv_cache, kv_page_indptr, kv_page_indices,
           kv_last_page_len):
    batch, num_q_heads, _, head_dim = q.shape
    q2 = q.reshape(batch, num_q_heads, head_dim) * (1.0 / math.sqrt(head_dim))

    grid_spec = pltpu.PrefetchScalarGridSpec(
        num_scalar_prefetch=3,
        grid=(1,),
        in_specs=[
            pl.BlockSpec(memory_space=pltpu.MemorySpace.VMEM),
            pl.BlockSpec(memory_space=pltpu.MemorySpace.HBM),
        ],
        out_specs=pl.BlockSpec(memory_space=pltpu.MemorySpace.VMEM),
        scratch_shapes=[
            pltpu.MemorySpace.SMEM((MAX_CHUNKS,), jnp.int32),
            pltpu.MemorySpace.SMEM((MAX_CHUNKS,), jnp.int32),
            pltpu.MemorySpace.VMEM(
                (NSLOTS, PAGES_PER_CHUNK, 2, NUM_KV_HEADS, PAGE_SIZE, HEAD_DIM),
                jnp.float32),
            pltpu.MemorySpace.VMEM((NUM_Q_HEADS, CHUNK_TOKENS), jnp.float32),
            pltpu.MemorySpace.VMEM((NUM_Q_HEADS, HEAD_DIM), jnp.float32),
            pltpu.MemorySpace.VMEM((NUM_Q_HEADS, 128), jnp.float32),
            pltpu.MemorySpace.VMEM((NUM_Q_HEADS, 128), jnp.float32),
            pltpu.MemorySpace.VMEM((NUM_Q_HEADS, HEAD_DIM), jnp.float32),
            pltpu.SemaphoreType.DMA((NSLOTS, PAGES_PER_CHUNK)),
        ],
    )
    out = pl.pallas_call(
        _attn_kernel,
        grid_spec=grid_spec,
        out_shape=jax.ShapeDtypeStruct((batch, num_q_heads, head_dim),
                                       jnp.float32),
    )(kv_page_indptr, kv_page_indices, kv_last_page_len,
      q2, paged_kv_cache)
    return out.reshape(batch, num_q_heads, 1, head_dim)
